# Initial kernel scaffold; baseline (speedup 1.0000x reference)
import jax
import jax.numpy as jnp
from jax.experimental import pallas as pl


def kernel(x, gemm, W, b, gamma, beta):
    raise NotImplementedError("write your pallas kernel here")


# speedup vs baseline: 2.3078x; 2.3078x over previous
import functools

import jax
import jax.numpy as jnp
from jax import lax
from jax.experimental import pallas as pl
from jax.experimental.pallas import tpu as pltpu
from jax.experimental.pallas import tpu_sc as plsc

_NTAP = 4
_NW = 32
_KC = 80
_EB = 2000


def _sc_gather(table, idx):
    n, = idx.shape
    _, c = table.shape
    per_w = n // _NW
    nchunk = per_w // _KC

    mesh = plsc.VectorSubcoreMesh(core_axis_name="c", subcore_axis_name="s")

    @functools.partial(
        pl.kernel,
        mesh=mesh,
        out_type=jax.ShapeDtypeStruct((n, c), table.dtype),
        scratch_types=[
            pltpu.VMEM((_KC,), jnp.int32),
            pltpu.VMEM((_KC, c), table.dtype),
            pltpu.SemaphoreType.DMA,
        ],
    )
    def gather_kernel(table_hbm, idx_hbm, out_hbm, idx_v, rows_v, sem):
        wid = lax.axis_index("s") * 2 + lax.axis_index("c")
        base_w = wid * per_w

        def chunk_step(t, carry):
            base = base_w + t * _KC
            pltpu.sync_copy(idx_hbm.at[pl.ds(base, _KC)], idx_v)
            pltpu.async_copy(table_hbm.at[idx_v], rows_v, sem).wait()
            pltpu.sync_copy(rows_v, out_hbm.at[pl.ds(base, _KC)])
            return carry

        lax.fori_loop(0, nchunk, chunk_step, 0)

    return gather_kernel(table, idx)


def _tc_conv_stats(xt, taps, wc):
    e, c = xt.shape

    def body(xt_ref, taps_ref, wc_ref, y_ref, s1_ref, s2_ref):
        i = pl.program_id(0)
        x = xt_ref[...]
        a = taps_ref[0]
        bb = taps_ref[1]
        cc = taps_ref[2]
        dd = taps_ref[3]
        h = jnp.concatenate(
            [x, a + cc, bb + dd, jnp.abs(a - cc), jnp.abs(bb - dd)], axis=1)
        y = jnp.dot(h, wc_ref[...], preferred_element_type=jnp.float32)
        y_ref[...] = y

        @pl.when(i == 0)
        def _init():
            s1_ref[...] = jnp.zeros_like(s1_ref)
            s2_ref[...] = jnp.zeros_like(s2_ref)

        s1_ref[...] += jnp.sum(y, axis=0, keepdims=True)
        s2_ref[...] += jnp.sum(y * y, axis=0, keepdims=True)

    return pl.pallas_call(
        body,
        grid=(e // _EB,),
        in_specs=[
            pl.BlockSpec((_EB, c), lambda i: (i, 0)),
            pl.BlockSpec((_NTAP, _EB, c), lambda i: (0, i, 0)),
            pl.BlockSpec((5 * c, c), lambda i: (0, 0)),
        ],
        out_specs=[
            pl.BlockSpec((_EB, c), lambda i: (i, 0)),
            pl.BlockSpec((1, c), lambda i: (0, 0)),
            pl.BlockSpec((1, c), lambda i: (0, 0)),
        ],
        out_shape=[
            jax.ShapeDtypeStruct((e, c), jnp.float32),
            jax.ShapeDtypeStruct((1, c), jnp.float32),
            jax.ShapeDtypeStruct((1, c), jnp.float32),
        ],
    )(xt, taps, wc)


def _tc_bn_relu(y, scale, shift):
    e, c = y.shape

    def body(y_ref, sc_ref, sh_ref, o_ref):
        o_ref[...] = jnp.maximum(y_ref[...] * sc_ref[...] + sh_ref[...], 0.0)

    return pl.pallas_call(
        body,
        grid=(e // _EB,),
        in_specs=[
            pl.BlockSpec((_EB, c), lambda i: (i, 0)),
            pl.BlockSpec((1, c), lambda i: (0, 0)),
            pl.BlockSpec((1, c), lambda i: (0, 0)),
        ],
        out_specs=pl.BlockSpec((_EB, c), lambda i: (i, 0)),
        out_shape=jax.ShapeDtypeStruct((e, c), jnp.float32),
    )(y, scale, shift)


def kernel(x, gemm, W, b, gamma, beta):
    _, c_in, e = x.shape
    c_out = W.shape[0]

    xt = jnp.swapaxes(x[0], 0, 1)
    idx = jnp.swapaxes(gemm[0], 0, 1).reshape(-1)
    taps = _sc_gather(xt, idx).reshape(_NTAP, e, c_in)

    wc = jnp.transpose(W, (2, 1, 0)).reshape(5 * c_in, c_out)
    y, s1, s2 = _tc_conv_stats(xt, taps, wc)

    mean = s1[0] / e
    var = s2[0] / e - mean * mean
    inv = gamma / jnp.sqrt(var + 1e-5)
    scale = inv[None]
    shift = (beta - mean * inv)[None]

    z = _tc_bn_relu(y, scale, shift)
    out = jnp.swapaxes(z, 0, 1)[None]
    return (out, gemm)


# speedup vs baseline: 3.2152x; 1.3932x over previous
import functools

import jax
import jax.numpy as jnp
from jax import lax
from jax.experimental import pallas as pl
from jax.experimental.pallas import tpu as pltpu
from jax.experimental.pallas import tpu_sc as plsc

_NTAP = 4
_NW = 32
_KC = 80
_EB = 2000
_EB2 = 3200


def _sc_gather(table, idx):
    n, = idx.shape
    _, c = table.shape
    per_w = n // _NW
    nchunk = per_w // _KC

    mesh = plsc.VectorSubcoreMesh(core_axis_name="c", subcore_axis_name="s")

    @functools.partial(
        pl.kernel,
        mesh=mesh,
        out_type=jax.ShapeDtypeStruct((n, c), table.dtype),
        scratch_types=[
            pltpu.VMEM((2, _KC), jnp.int32),
            pltpu.VMEM((2, _KC, c), table.dtype),
            pltpu.SemaphoreType.DMA,
            pltpu.SemaphoreType.DMA,
            pltpu.SemaphoreType.DMA,
        ],
    )
    def gather_kernel(table_hbm, idx_hbm, out_hbm, idx_v, rows_v,
                      sem_i, sem_g, sem_w):
        wid = lax.axis_index("s") * 2 + lax.axis_index("c")
        base_w = wid * per_w

        pltpu.async_copy(idx_hbm.at[pl.ds(base_w, _KC)], idx_v.at[0], sem_i)

        def chunk_step(t, carry):
            s = t % 2
            base = base_w + t * _KC
            pltpu.make_async_copy(
                idx_hbm.at[pl.ds(base, _KC)], idx_v.at[s], sem_i).wait()

            @pl.when(t + 1 < nchunk)
            def _prefetch():
                pltpu.async_copy(
                    idx_hbm.at[pl.ds(base + _KC, _KC)], idx_v.at[1 - s], sem_i)

            @pl.when(t >= 2)
            def _reclaim():
                pltpu.make_async_copy(
                    rows_v.at[s], out_hbm.at[pl.ds(base - 2 * _KC, _KC)],
                    sem_w).wait()

            pltpu.async_copy(table_hbm.at[idx_v.at[s]], rows_v.at[s],
                             sem_g).wait()
            pltpu.async_copy(rows_v.at[s], out_hbm.at[pl.ds(base, _KC)], sem_w)
            return carry

        lax.fori_loop(0, nchunk, chunk_step, 0)
        pltpu.make_async_copy(
            rows_v.at[0], out_hbm.at[pl.ds(base_w, _KC)], sem_w).wait()
        pltpu.make_async_copy(
            rows_v.at[0], out_hbm.at[pl.ds(base_w, _KC)], sem_w).wait()

    return gather_kernel(table, idx)


def _tc_conv_stats(xt, taps, wc):
    e, c = xt.shape

    def body(xt_ref, taps_ref, wc_ref, y_ref, s1_ref, s2_ref):
        i = pl.program_id(0)
        x = xt_ref[...]
        a = taps_ref[0]
        bb = taps_ref[1]
        cc = taps_ref[2]
        dd = taps_ref[3]
        h = jnp.concatenate(
            [x, a + cc, bb + dd, jnp.abs(a - cc), jnp.abs(bb - dd)], axis=1)
        y = jnp.dot(h, wc_ref[...], preferred_element_type=jnp.float32)
        y_ref[...] = y

        @pl.when(i == 0)
        def _init():
            s1_ref[...] = jnp.zeros_like(s1_ref)
            s2_ref[...] = jnp.zeros_like(s2_ref)

        s1_ref[...] += jnp.sum(y, axis=0, keepdims=True)
        s2_ref[...] += jnp.sum(y * y, axis=0, keepdims=True)

    return pl.pallas_call(
        body,
        grid=(e // _EB,),
        in_specs=[
            pl.BlockSpec((_EB, c), lambda i: (i, 0)),
            pl.BlockSpec((_NTAP, _EB, c), lambda i: (0, i, 0)),
            pl.BlockSpec((5 * c, c), lambda i: (0, 0)),
        ],
        out_specs=[
            pl.BlockSpec((_EB, c), lambda i: (i, 0)),
            pl.BlockSpec((1, c), lambda i: (0, 0)),
            pl.BlockSpec((1, c), lambda i: (0, 0)),
        ],
        out_shape=[
            jax.ShapeDtypeStruct((e, c), jnp.float32),
            jax.ShapeDtypeStruct((1, c), jnp.float32),
            jax.ShapeDtypeStruct((1, c), jnp.float32),
        ],
    )(xt, taps, wc)


def _tc_bn_relu(y, scale, shift):
    e, c = y.shape

    def body(y_ref, sc_ref, sh_ref, o_ref):
        z = jnp.maximum(y_ref[...] * sc_ref[...] + sh_ref[...], 0.0)
        o_ref[...] = z.T

    return pl.pallas_call(
        body,
        grid=(e // _EB2,),
        in_specs=[
            pl.BlockSpec((_EB2, c), lambda i: (i, 0)),
            pl.BlockSpec((1, c), lambda i: (0, 0)),
            pl.BlockSpec((1, c), lambda i: (0, 0)),
        ],
        out_specs=pl.BlockSpec((c, _EB2), lambda i: (0, i)),
        out_shape=jax.ShapeDtypeStruct((c, e), jnp.float32),
    )(y, scale, shift)


def kernel(x, gemm, W, b, gamma, beta):
    _, c_in, e = x.shape
    c_out = W.shape[0]

    xt = jnp.swapaxes(x[0], 0, 1)
    idx = jnp.swapaxes(gemm[0], 0, 1).reshape(-1)
    taps = _sc_gather(xt, idx).reshape(_NTAP, e, c_in)

    wc = jnp.transpose(W, (2, 1, 0)).reshape(5 * c_in, c_out)
    y, s1, s2 = _tc_conv_stats(xt, taps, wc)

    mean = s1[0] / e
    var = s2[0] / e - mean * mean
    inv = gamma / jnp.sqrt(var + 1e-5)
    scale = inv[None]
    shift = (beta - mean * inv)[None]

    out = _tc_bn_relu(y, scale, shift)[None]
    return (out, gemm)
